# baseline (speedup 1.0000x reference)
import dataclasses

import jax
import jax.numpy as jnp
from jax import lax
from jax.experimental import pallas as pl
from jax.experimental.pallas import tpu as pltpu
from jax.experimental.pallas import tpu_sc as plsc

K = 100000
D = 128
Q = 16384

NC = 2
NS = 16
L = 16
NW = NC * NS
QPW = Q // NW
CHUNK = 64
NCHUNK = QPW // CHUNK
SEARCH_STEPS = 17


def _sc_kernel(xs_hbm, ys_hbm, x_hbm, out_hbm,
               xs_v, x_v, idx_v, buf0, buf1, sem_in, sem_g0, sem_g1):
    wid = lax.axis_index("s") * NC + lax.axis_index("c")
    base = wid * QPW

    with jax.named_scope("stage_xs"):
        nsplit = 4
        piece = K // nsplit
        hs = [
            pltpu.async_copy(
                xs_hbm.at[pl.ds(s * piece, piece)],
                xs_v.at[pl.ds(s * piece, piece)], sem_in)
            for s in range(nsplit)
        ]
        h_x = pltpu.async_copy(x_hbm.at[pl.ds(base, QPW)], x_v, sem_g0)
        h_x.wait()
        for h in hs:
            h.wait()

    @plsc.parallel_loop(0, QPW, step=L, unroll=4)
    def _(i):
      with jax.named_scope("search"):
        xq = x_v[pl.ds(i, L)]
        lo = jnp.zeros((L,), jnp.int32)
        hi = jnp.full((L,), K, jnp.int32)
        for _step in range(SEARCH_STEPS):
            mid = jnp.right_shift(lo + hi, 1)
            mid_safe = jnp.minimum(mid, K - 1)
            xv = plsc.load_gather(xs_v, [mid_safe])
            valid = lo < hi
            le = xv <= xq
            lo = jnp.where(valid & le, mid + 1, lo)
            hi = jnp.where(valid & jnp.logical_not(le), mid, hi)
        row = jnp.where(lo == 0, K - 1, lo - 1)
        idx_v[pl.ds(i, L)] = row

    bufs = (buf0, buf1)
    sems = (sem_g0, sem_g1)

    def start(c):
        return pltpu.async_copy(
            ys_hbm.at[idx_v.at[pl.ds(c * CHUNK, CHUNK)]], bufs[c % 2], sems[c % 2])

    with jax.named_scope("rowgather"):
        h_next = start(0)
        for c in range(NCHUNK):
            h = h_next
            if c + 1 < NCHUNK:
                h_next = start(c + 1)
            h.wait()
            pltpu.sync_copy(bufs[c % 2], out_hbm.at[pl.ds(base + c * CHUNK, CHUNK)])


def kernel(xs, ys, x):
    mesh = plsc.VectorSubcoreMesh(core_axis_name="c", subcore_axis_name="s")
    cp = pltpu.CompilerParams()
    if "needs_layout_passes" in pltpu.CompilerParams.__dataclass_fields__:
        cp = dataclasses.replace(cp, needs_layout_passes=False)
    run = pl.kernel(
        _sc_kernel,
        out_type=jax.ShapeDtypeStruct((Q, D), jnp.float32),
        mesh=mesh,
        scratch_types=[
            pltpu.VMEM((K,), jnp.float32),
            pltpu.VMEM((QPW,), jnp.float32),
            pltpu.VMEM((QPW,), jnp.int32),
            pltpu.VMEM((CHUNK, D), jnp.float32),
            pltpu.VMEM((CHUNK, D), jnp.float32),
            pltpu.SemaphoreType.DMA,
            pltpu.SemaphoreType.DMA,
            pltpu.SemaphoreType.DMA,
        ],
        compiler_params=cp,
    )
    return run(xs, ys, x)


# speedup vs baseline: 1.0613x; 1.0613x over previous
import dataclasses

import jax
import jax.numpy as jnp
from jax import lax
from jax.experimental import pallas as pl
from jax.experimental.pallas import tpu as pltpu
from jax.experimental.pallas import tpu_sc as plsc

K = 100000
D = 128
Q = 16384

NC = 2
NS = 16
L = 16
NW = NC * NS
QPW = Q // NW
CHUNK = 64
NCHUNK = QPW // CHUNK
NBUF = 3
SEARCH_STEPS = 17
NSPLIT = 10
PIECE = K // NSPLIT


def _sc_kernel(xs_hbm, ys_hbm, x_hbm, out_hbm,
               xs_v, x_v, idx_v, bufs, sem_in, sems_g, sems_w):
    wid = lax.axis_index("s") * NC + lax.axis_index("c")
    base = wid * QPW

    hs = []
    for s in range(NSPLIT):
        start = lax.rem((wid + s) * PIECE, K)
        hs.append(pltpu.async_copy(
            xs_hbm.at[pl.ds(start, PIECE)], xs_v.at[pl.ds(start, PIECE)],
            sem_in))
    h_x = pltpu.async_copy(x_hbm.at[pl.ds(base, QPW)], x_v, sem_in)
    h_x.wait()
    for h in hs:
        h.wait()

    def search_chunk(c):
        @plsc.parallel_loop(c * CHUNK, (c + 1) * CHUNK, step=L, unroll=4)
        def _(i):
            xq = x_v[pl.ds(i, L)]
            lo = jnp.zeros((L,), jnp.int32)
            hi = jnp.full((L,), K, jnp.int32)
            for _step in range(SEARCH_STEPS):
                mid = jnp.right_shift(lo + hi, 1)
                mid_safe = jnp.minimum(mid, K - 1)
                xv = plsc.load_gather(xs_v, [mid_safe])
                valid = lo < hi
                le = xv <= xq
                lo = jnp.where(valid & le, mid + 1, lo)
                hi = jnp.where(valid & jnp.logical_not(le), mid, hi)
            row = jnp.where(lo == 0, K - 1, lo - 1)
            idx_v[pl.ds(i, L)] = row

    def fire_gather(c):
        return pltpu.async_copy(
            ys_hbm.at[idx_v.at[pl.ds(c * CHUNK, CHUNK)]],
            bufs.at[c % NBUF], sems_g.at[c % NBUF])

    def fire_writeout(c):
        return pltpu.async_copy(
            bufs.at[c % NBUF], out_hbm.at[pl.ds(base + c * CHUNK, CHUNK)],
            sems_w.at[c % NBUF])

    gathers = [None] * NCHUNK
    writes = [None] * NCHUNK
    for c in range(NCHUNK):
        search_chunk(c)
        if c >= NBUF:
            writes[c - NBUF].wait()
        gathers[c] = fire_gather(c)
        if c >= 1:
            gathers[c - 1].wait()
            writes[c - 1] = fire_writeout(c - 1)
    gathers[NCHUNK - 1].wait()
    writes[NCHUNK - 1] = fire_writeout(NCHUNK - 1)
    for c in range(NCHUNK - NBUF, NCHUNK):
        writes[c].wait()


def kernel(xs, ys, x):
    mesh = plsc.VectorSubcoreMesh(core_axis_name="c", subcore_axis_name="s")
    cp = pltpu.CompilerParams()
    if "needs_layout_passes" in pltpu.CompilerParams.__dataclass_fields__:
        cp = dataclasses.replace(cp, needs_layout_passes=False)
    run = pl.kernel(
        _sc_kernel,
        out_type=jax.ShapeDtypeStruct((Q, D), jnp.float32),
        mesh=mesh,
        scratch_types=[
            pltpu.VMEM((K,), jnp.float32),
            pltpu.VMEM((QPW,), jnp.float32),
            pltpu.VMEM((QPW,), jnp.int32),
            pltpu.VMEM((NBUF, CHUNK, D), jnp.float32),
            pltpu.SemaphoreType.DMA,
            pltpu.SemaphoreType.DMA((NBUF,)),
            pltpu.SemaphoreType.DMA((NBUF,)),
        ],
        compiler_params=cp,
    )
    return run(xs, ys, x)


# speedup vs baseline: 1.0664x; 1.0048x over previous
import dataclasses

import jax
import jax.numpy as jnp
from jax import lax
from jax.experimental import pallas as pl
from jax.experimental.pallas import tpu as pltpu
from jax.experimental.pallas import tpu_sc as plsc

K = 100000
D = 128
Q = 16384

NC = 2
NS = 16
L = 16
NW = NC * NS
QPW = Q // NW
CHUNK = 64
NCHUNK = QPW // CHUNK
NBUF = 3
SEARCH_STEPS = 17
NSPLIT = 10
PIECE = K // NSPLIT


def _sc_kernel(xs_hbm, ys_hbm, x_hbm, out_hbm,
               xs_v, x_v, idx_v, bufs, sem_in, sems_g, sems_w):
    wid = lax.axis_index("s") * NC + lax.axis_index("c")
    base = wid * QPW

    hs = []
    sc1 = jax.named_scope("stage_xs")
    sc1.__enter__()
    for s in range(NSPLIT):
        start = lax.rem((wid + s) * PIECE, K)
        hs.append(pltpu.async_copy(
            xs_hbm.at[pl.ds(start, PIECE)], xs_v.at[pl.ds(start, PIECE)],
            sem_in))
    h_x = pltpu.async_copy(x_hbm.at[pl.ds(base, QPW)], x_v, sem_in)
    h_x.wait()
    for h in hs:
        h.wait()
    sc1.__exit__(None, None, None)

    def search_chunk(c):
        @plsc.parallel_loop(c * CHUNK, (c + 1) * CHUNK, step=L, unroll=4)
        def _(i):
            xq = x_v[pl.ds(i, L)]
            lo = jnp.zeros((L,), jnp.int32)
            hi = jnp.full((L,), K, jnp.int32)
            for _step in range(SEARCH_STEPS):
                mid = jnp.right_shift(lo + hi, 1)
                mid_safe = jnp.minimum(mid, K - 1)
                xv = plsc.load_gather(xs_v, [mid_safe])
                valid = lo < hi
                le = xv <= xq
                lo = jnp.where(valid & le, mid + 1, lo)
                hi = jnp.where(valid & jnp.logical_not(le), mid, hi)
            row = jnp.where(lo == 0, K - 1, lo - 1)
            idx_v[pl.ds(i, L)] = row

    def fire_gather(c):
        return pltpu.async_copy(
            ys_hbm.at[idx_v.at[pl.ds(c * CHUNK, CHUNK)]],
            bufs.at[c % NBUF], sems_g.at[c % NBUF])

    def fire_writeout(c):
        return pltpu.async_copy(
            bufs.at[c % NBUF], out_hbm.at[pl.ds(base + c * CHUNK, CHUNK)],
            sems_w.at[c % NBUF])

    sc2 = jax.named_scope("pipeline")
    sc2.__enter__()
    gathers = [None] * NCHUNK
    writes = [None] * NCHUNK
    for c in range(NCHUNK):
        search_chunk(c)
        if c >= NBUF:
            writes[c - NBUF].wait()
        gathers[c] = fire_gather(c)
        if c >= 1:
            gathers[c - 1].wait()
            writes[c - 1] = fire_writeout(c - 1)
    gathers[NCHUNK - 1].wait()
    writes[NCHUNK - 1] = fire_writeout(NCHUNK - 1)
    for c in range(NCHUNK - NBUF, NCHUNK):
        writes[c].wait()
    sc2.__exit__(None, None, None)


def kernel(xs, ys, x):
    mesh = plsc.VectorSubcoreMesh(core_axis_name="c", subcore_axis_name="s")
    cp = pltpu.CompilerParams()
    if "needs_layout_passes" in pltpu.CompilerParams.__dataclass_fields__:
        cp = dataclasses.replace(cp, needs_layout_passes=False)
    run = pl.kernel(
        _sc_kernel,
        out_type=jax.ShapeDtypeStruct((Q, D), jnp.float32),
        mesh=mesh,
        scratch_types=[
            pltpu.VMEM((K,), jnp.float32),
            pltpu.VMEM((QPW,), jnp.float32),
            pltpu.VMEM((QPW,), jnp.int32),
            pltpu.VMEM((NBUF, CHUNK, D), jnp.float32),
            pltpu.SemaphoreType.DMA,
            pltpu.SemaphoreType.DMA((NBUF,)),
            pltpu.SemaphoreType.DMA((NBUF,)),
        ],
        compiler_params=cp,
    )
    return run(xs, ys, x)


# speedup vs baseline: 1.0888x; 1.0209x over previous
import dataclasses

import jax
import jax.numpy as jnp
from jax import lax
from jax.experimental import pallas as pl
from jax.experimental.pallas import tpu as pltpu
from jax.experimental.pallas import tpu_sc as plsc

K = 100000
D = 128
Q = 16384

NC = 2
NS = 16
L = 16
NW = NC * NS
QPW = Q // NW
CHUNK = 64
NCHUNK = QPW // CHUNK
SEARCH_STEPS = 17
NSPLIT = 10
PIECE = K // NSPLIT


def _sc_kernel(xs_hbm, ys_hbm, x_hbm, out_hbm,
               xs_v, x_v, idx_v, bufs, sem_in, sems_g, sems_w):
    cid = lax.axis_index("c")
    sid = lax.axis_index("s")
    wid = sid * NC + cid
    base = wid * QPW

    hs = []
    for s in range(NSPLIT):
        start = lax.rem((wid + s) * PIECE, K)
        hs.append(pltpu.async_copy(
            xs_hbm.at[pl.ds(start, PIECE)], xs_v.at[pl.ds(start, PIECE)],
            sem_in))
    h_x = pltpu.async_copy(x_hbm.at[pl.ds(base, QPW)], x_v, sem_in)
    h_x.wait()
    for h in hs:
        h.wait()

    def search_chunk(off):
        @plsc.parallel_loop(0, CHUNK, step=L, unroll=4)
        def _(i):
            xq = x_v[pl.ds(off + i, L)]
            idx = jnp.zeros((L,), jnp.int32)
            bit = 1 << (SEARCH_STEPS - 1)
            for _step in range(SEARCH_STEPS):
                cand = idx + bit
                gidx = jnp.minimum(cand, K) - 1
                xv = plsc.load_gather(xs_v, [gidx])
                ok = (cand <= K) & (xv <= xq)
                idx = jnp.where(ok, cand, idx)
                bit >>= 1
            row = jnp.where(idx == 0, K - 1, idx - 1)
            idx_v[pl.ds(off + i, L)] = row

    def fire_gather(buf, b, off):
        return pltpu.async_copy(
            ys_hbm.at[idx_v.at[pl.ds(off, CHUNK)]], buf, sems_g.at[b])

    def wait_gather(buf, b, off):
        pltpu.make_async_copy(
            ys_hbm.at[idx_v.at[pl.ds(off, CHUNK)]], buf, sems_g.at[b]).wait()

    def fire_write(buf, b, off):
        return pltpu.async_copy(
            buf, out_hbm.at[pl.ds(base + off, CHUNK)], sems_w.at[b])

    def wait_write(buf, b, off):
        pltpu.make_async_copy(
            buf, out_hbm.at[pl.ds(base + off, CHUNK)], sems_w.at[b]).wait()

    buf0 = bufs.at[0]
    buf1 = bufs.at[1]

    @pl.loop(0, NCHUNK // 2)
    def _(p):
        c0 = 2 * p * CHUNK
        c1 = c0 + CHUNK
        search_chunk(c0)

        @pl.when(p > 0)
        def _():
            wait_gather(buf1, 1, c0 - CHUNK)
            fire_write(buf1, 1, c0 - CHUNK)
            wait_write(buf0, 0, c0 - 2 * CHUNK)
        fire_gather(buf0, 0, c0)
        search_chunk(c1)
        wait_gather(buf0, 0, c0)
        fire_write(buf0, 0, c0)

        @pl.when(p > 0)
        def _():
            wait_write(buf1, 1, c1 - 2 * CHUNK)
        fire_gather(buf1, 1, c1)

    last = (NCHUNK - 1) * CHUNK
    wait_gather(buf1, 1, last)
    fire_write(buf1, 1, last)
    wait_write(buf0, 0, last - CHUNK)
    wait_write(buf1, 1, last)


def kernel(xs, ys, x):
    mesh = plsc.VectorSubcoreMesh(core_axis_name="c", subcore_axis_name="s")
    cp = pltpu.CompilerParams()
    if "needs_layout_passes" in pltpu.CompilerParams.__dataclass_fields__:
        cp = dataclasses.replace(cp, needs_layout_passes=False)
    run = pl.kernel(
        _sc_kernel,
        out_type=jax.ShapeDtypeStruct((Q, D), jnp.float32),
        mesh=mesh,
        scratch_types=[
            pltpu.VMEM((K,), jnp.float32),
            pltpu.VMEM((QPW,), jnp.float32),
            pltpu.VMEM((QPW,), jnp.int32),
            pltpu.VMEM((2, CHUNK, D), jnp.float32),
            pltpu.SemaphoreType.DMA,
            pltpu.SemaphoreType.DMA((2,)),
            pltpu.SemaphoreType.DMA((2,)),
        ],
        compiler_params=cp,
    )
    return run(xs, ys, x)


# speedup vs baseline: 1.2367x; 1.1358x over previous
import dataclasses

import jax
import jax.numpy as jnp
from jax import lax
from jax.experimental import pallas as pl
from jax.experimental.pallas import tpu as pltpu
from jax.experimental.pallas import tpu_sc as plsc

K = 100000
D = 128
Q = 16384

NC = 2
NS = 16
L = 16
NW = NC * NS
QPW = Q // NW
CHUNK = 64
NCHUNK = QPW // CHUNK
W = 32
KL1 = K // W
L1_STEPS = 12
L2_STEPS = 6
L1PW = 224
L1PAD = L1PW * NS
SRCW = L1PW * W


def _search_l1(x_v, l1_v, rvec_v, off):
    zvec = jnp.zeros((L,), jnp.int32)

    @plsc.parallel_loop(0, CHUNK, step=L, unroll=4)
    def _(i):
        xq = x_v[pl.ds(off + i, L)]
        c1 = jnp.zeros((L,), jnp.int32)
        bit = 1 << (L1_STEPS - 1)
        for _step in range(L1_STEPS):
            cand = c1 + bit
            gidx = jnp.minimum(cand, L1PAD) - 1
            xv = plsc.load_gather(l1_v, [gidx])
            ok = xv <= xq
            c1 = jnp.where(ok, cand, c1)
            bit >>= 1
        rvec_v[pl.ds(off + i, L)] = jnp.maximum(c1 - 1, 0)


def _search_l2(x_v, rvec_v, win, idx_v, off):
    lanes = jnp.arange(L, dtype=jnp.int32)

    @plsc.parallel_loop(0, CHUNK, step=L, unroll=4)
    def _(i):
        xq = x_v[pl.ds(off + i, L)]
        r = rvec_v[pl.ds(off + i, L)]
        qvec = lanes + i
        cnt = jnp.zeros((L,), jnp.int32)
        bit = 1 << (L2_STEPS - 1)
        for _step in range(L2_STEPS):
            cand = cnt + bit
            gidx = jnp.minimum(cand, W) - 1
            wv = plsc.load_gather(win, [qvec, gidx])
            ok = (cand <= W) & (wv <= xq)
            cnt = jnp.where(ok, cand, cnt)
            bit >>= 1
        idx = r * W + cnt
        idx_v[pl.ds(off + i, L)] = jnp.where(idx == 0, K - 1, idx - 1)


def _sc_kernel(xsw_hbm, ys_hbm, x_hbm, out_hbm,
               l1_v, stage_v, l1_loc_v, x_v, rvec_v, idx_v, wins, bufs, sh_l1,
               sem_in, sems_win, sems_g, sems_w):
    cid = lax.axis_index("c")
    sid = lax.axis_index("s")
    wid = sid * NC + cid
    base = wid * QPW

    h_x = pltpu.async_copy(x_hbm.at[pl.ds(base, QPW)], x_v, sem_in)
    j0 = sid * L1PW
    pltpu.sync_copy(xsw_hbm.at[pl.ds(j0, L1PW), :], stage_v)
    zv = jnp.zeros((L,), jnp.int32)
    lanes0 = jnp.arange(L, dtype=jnp.int32)

    @plsc.parallel_loop(0, L1PW, step=L, unroll=4)
    def _(i):
        l1_loc_v[pl.ds(i, L)] = plsc.load_gather(stage_v, [i + lanes0, zv])

    pltpu.sync_copy(l1_loc_v, sh_l1.at[pl.ds(j0, L1PW)])
    plsc.subcore_barrier()
    pltpu.async_copy(sh_l1, l1_v, sem_in).wait()
    h_x.wait()

    def fire_win(b, off):
        return pltpu.async_copy(
            xsw_hbm.at[rvec_v.at[pl.ds(off, CHUNK)]], wins.at[b],
            sems_win.at[b])

    def wait_win(b, off):
        pltpu.make_async_copy(
            xsw_hbm.at[rvec_v.at[pl.ds(off, CHUNK)]], wins.at[b],
            sems_win.at[b]).wait()

    def fire_gather(b, off):
        return pltpu.async_copy(
            ys_hbm.at[idx_v.at[pl.ds(off, CHUNK)]], bufs.at[b], sems_g.at[b])

    def wait_gather(b, off):
        pltpu.make_async_copy(
            ys_hbm.at[idx_v.at[pl.ds(off, CHUNK)]], bufs.at[b],
            sems_g.at[b]).wait()

    def fire_write(b, off):
        return pltpu.async_copy(
            bufs.at[b], out_hbm.at[pl.ds(base + off, CHUNK)], sems_w.at[b])

    def wait_write(b, off):
        pltpu.make_async_copy(
            bufs.at[b], out_hbm.at[pl.ds(base + off, CHUNK)], sems_w.at[b]).wait()

    _search_l1(x_v, l1_v, rvec_v, 0)
    fire_win(0, 0)

    @pl.loop(0, NCHUNK // 2)
    def _(p):
        c0 = 2 * p * CHUNK
        c1 = c0 + CHUNK

        @pl.when(p > 0)
        def _():
            wait_gather(1, c0 - CHUNK)
            fire_write(1, c0 - CHUNK)
        _search_l1(x_v, l1_v, rvec_v, c1)
        fire_win(1, c1)
        wait_win(0, c0)
        _search_l2(x_v, rvec_v, wins.at[0], idx_v, c0)

        @pl.when(p > 0)
        def _():
            wait_write(0, c0 - 2 * CHUNK)
        fire_gather(0, c0)

        @pl.when(p < NCHUNK // 2 - 1)
        def _():
            _search_l1(x_v, l1_v, rvec_v, c1 + CHUNK)
            fire_win(0, c1 + CHUNK)
        wait_win(1, c1)
        _search_l2(x_v, rvec_v, wins.at[1], idx_v, c1)
        wait_gather(0, c0)
        fire_write(0, c0)

        @pl.when(p > 0)
        def _():
            wait_write(1, c1 - 2 * CHUNK)
        fire_gather(1, c1)

    last = (NCHUNK - 1) * CHUNK
    wait_gather(1, last)
    fire_write(1, last)
    wait_write(0, last - CHUNK)
    wait_write(1, last)


def kernel(xs, ys, x):
    mesh = plsc.VectorSubcoreMesh(core_axis_name="c", subcore_axis_name="s")
    cp = pltpu.CompilerParams()
    if "needs_layout_passes" in pltpu.CompilerParams.__dataclass_fields__:
        cp = dataclasses.replace(cp, needs_layout_passes=False)
    if "use_tc_tiling_on_sc" in pltpu.CompilerParams.__dataclass_fields__:
        cp = dataclasses.replace(cp, use_tc_tiling_on_sc=False)
    run = pl.kernel(
        _sc_kernel,
        out_type=jax.ShapeDtypeStruct((Q, D), jnp.float32),
        mesh=mesh,
        scratch_types=[
            pltpu.VMEM((L1PAD,), jnp.float32),
            pltpu.VMEM((L1PW, W), jnp.float32),
            pltpu.VMEM((L1PW,), jnp.float32),
            pltpu.VMEM((QPW,), jnp.float32),
            pltpu.VMEM((QPW,), jnp.int32),
            pltpu.VMEM((QPW,), jnp.int32),
            pltpu.VMEM((2, CHUNK, W), jnp.float32),
            pltpu.VMEM((2, CHUNK, D), jnp.float32),
            pltpu.VMEM_SHARED((L1PAD,), jnp.float32),
            pltpu.SemaphoreType.DMA,
            pltpu.SemaphoreType.DMA((2,)),
            pltpu.SemaphoreType.DMA((2,)),
            pltpu.SemaphoreType.DMA((2,)),
        ],
        compiler_params=cp,
    )
    xs_pad = jnp.concatenate(
        [xs, jnp.full((L1PAD * W - K,), jnp.inf, jnp.float32)])
    return run(xs_pad.reshape(L1PAD, W), ys, x)
